# Initial kernel scaffold; baseline (speedup 1.0000x reference)
import jax
import jax.numpy as jnp
from jax.experimental import pallas as pl


def kernel(y, edge_index, edge_weight, W0_0, W1_0, b_0, W0_1, W1_1, b_1):
    raise NotImplementedError("write your pallas kernel here")


# speedup vs baseline: 4.5688x; 4.5688x over previous
import functools

import jax
import jax.numpy as jnp
from jax import lax
from jax.experimental import pallas as pl
from jax.experimental.pallas import tpu as pltpu
from jax.experimental.pallas import tpu_sc as plsc

N = 10000
E = 320000
D = 128

NC = 2
NS = 16
NW = NC * NS

C = 128
G = E // C
NP = 10240
ROWS_PER_TILE = NP // NS

_slope = 0.01

_GATHER_DNUMS = lax.GatherDimensionNumbers(
    offset_dims=(), collapsed_slice_dims=(0,), start_index_map=(0,))


def _lane_bcast(vec, lane):
    idx = jnp.full((16, 1), lane, jnp.int32)
    return lax.gather(vec, idx, dimension_numbers=_GATHER_DNUMS,
                      slice_sizes=(1,),
                      mode=lax.GatherScatterMode.PROMISE_IN_BOUNDS)


def _sc_agg_body(z_hbm, src_hbm, dst_hbm, ew_hbm, out_hbm,
                 src_v, dst_v, ew_v, rows_v, acc_sh, sem):
    cid = lax.axis_index("c")
    sid = lax.axis_index("s")
    wid = sid * NC + cid

    if True:
        def _zrow(i, _):
            for j in range(D // 16):
                rows_v[i, pl.ds(j * 16, 16)] = jnp.zeros((16,), jnp.float32)
            return 0
        lax.fori_loop(0, C, _zrow, 0)
        zbase = sid * ROWS_PER_TILE
        for kz in range(ROWS_PER_TILE // C):
            pltpu.sync_copy(rows_v,
                            acc_sh.at[pl.ds(zbase + kz * C, C)])
        plsc.subcore_barrier()

        n_chunks = (G - wid + NW - 1) // NW

        def _chunk(k, _):
            base = (wid + k * NW) * C
            pltpu.sync_copy(src_hbm.at[pl.ds(base, C)], src_v)
            pltpu.sync_copy(dst_hbm.at[pl.ds(base, C)], dst_v)
            pltpu.sync_copy(ew_hbm.at[pl.ds(base, C)], ew_v)
            pltpu.async_copy(z_hbm.at[src_v], rows_v, sem).wait()

            def _edge16(g, _):
                wvec = ew_v[pl.ds(g * 16, 16)]
                for lane in range(16):
                    bw = _lane_bcast(wvec, lane)
                    e = g * 16 + lane
                    for j in range(D // 16):
                        sl = pl.ds(j * 16, 16)
                        rows_v[e, sl] = rows_v[e, sl] * bw
                return 0
            lax.fori_loop(0, C // 16, _edge16, 0)

            pltpu.sync_copy(rows_v, acc_sh.at[dst_v], add=True)
            return 0
        lax.fori_loop(0, n_chunks, _chunk, 0)
        plsc.subcore_barrier()

        pltpu.sync_copy(acc_sh.at[pl.ds(zbase, ROWS_PER_TILE)],
                        out_hbm.at[cid, pl.ds(zbase, ROWS_PER_TILE)])


@functools.partial(jax.jit, static_argnames=())
def _sc_agg(z, src, dst, ew):
    mesh = plsc.VectorSubcoreMesh(core_axis_name="c", subcore_axis_name="s")
    f = pl.kernel(
        _sc_agg_body,
        out_type=jax.ShapeDtypeStruct((NC, NP, D), jnp.float32),
        mesh=mesh,
        scratch_types=[
            pltpu.VMEM((C,), jnp.int32),
            pltpu.VMEM((C,), jnp.int32),
            pltpu.VMEM((C,), jnp.float32),
            pltpu.VMEM((C, D), jnp.float32),
            pltpu.VMEM_SHARED((NP, D), jnp.float32),
            pltpu.SemaphoreType.DMA,
        ],
    )
    return f(z, src, dst, ew)


_BN = 1000


def _tc_pre_body(x_ref, w0t_ref, w1t_ref, b_ref, a_ref, z_ref):
    x = x_ref[...]
    a_ref[...] = (jnp.dot(x, w0t_ref[...], preferred_element_type=jnp.float32,
                          precision=lax.Precision.HIGHEST)
                  + b_ref[...] + x)
    z_ref[...] = jnp.dot(x, w1t_ref[...], preferred_element_type=jnp.float32,
                         precision=lax.Precision.HIGHEST)


def _tc_pre(x, w0t, w1t, b):
    grid = (N // _BN,)
    return pl.pallas_call(
        _tc_pre_body,
        grid=grid,
        in_specs=[
            pl.BlockSpec((_BN, D), lambda i: (i, 0)),
            pl.BlockSpec((D, D), lambda i: (0, 0)),
            pl.BlockSpec((D, D), lambda i: (0, 0)),
            pl.BlockSpec((1, D), lambda i: (0, 0)),
        ],
        out_specs=[
            pl.BlockSpec((_BN, D), lambda i: (i, 0)),
            pl.BlockSpec((_BN, D), lambda i: (i, 0)),
        ],
        out_shape=[
            jax.ShapeDtypeStruct((N, D), jnp.float32),
            jax.ShapeDtypeStruct((N, D), jnp.float32),
        ],
    )(x, w0t, w1t, b)


def _tc_mid_body(a_ref, agg_ref, w0t_ref, w1t_ref, b_ref, a_out_ref, z_out_ref):
    h = a_ref[...] + agg_ref[0] + agg_ref[1]
    y = jnp.where(h >= 0, h, _slope * h)
    a_out_ref[...] = (jnp.dot(y, w0t_ref[...], preferred_element_type=jnp.float32,
                              precision=lax.Precision.HIGHEST)
                      + b_ref[...] + y)
    z_out_ref[...] = jnp.dot(y, w1t_ref[...], preferred_element_type=jnp.float32,
                             precision=lax.Precision.HIGHEST)


def _tc_mid(a, agg, w0t, w1t, b):
    grid = (N // _BN,)
    return pl.pallas_call(
        _tc_mid_body,
        grid=grid,
        in_specs=[
            pl.BlockSpec((_BN, D), lambda i: (i, 0)),
            pl.BlockSpec((NC, _BN, D), lambda i: (0, i, 0)),
            pl.BlockSpec((D, D), lambda i: (0, 0)),
            pl.BlockSpec((D, D), lambda i: (0, 0)),
            pl.BlockSpec((1, D), lambda i: (0, 0)),
        ],
        out_specs=[
            pl.BlockSpec((_BN, D), lambda i: (i, 0)),
            pl.BlockSpec((_BN, D), lambda i: (i, 0)),
        ],
        out_shape=[
            jax.ShapeDtypeStruct((N, D), jnp.float32),
            jax.ShapeDtypeStruct((N, D), jnp.float32),
        ],
    )(a, agg, w0t, w1t, b)


def _tc_post_body(a_ref, agg_ref, y_ref):
    h = a_ref[...] + agg_ref[0] + agg_ref[1]
    y_ref[...] = jnp.where(h >= 0, h, _slope * h)


def _tc_post(a, agg):
    grid = (N // _BN,)
    return pl.pallas_call(
        _tc_post_body,
        grid=grid,
        in_specs=[
            pl.BlockSpec((_BN, D), lambda i: (i, 0)),
            pl.BlockSpec((NC, _BN, D), lambda i: (0, i, 0)),
        ],
        out_specs=pl.BlockSpec((_BN, D), lambda i: (i, 0)),
        out_shape=jax.ShapeDtypeStruct((N, D), jnp.float32),
    )(a, agg)


def kernel(y, edge_index, edge_weight, W0_0, W1_0, b_0, W0_1, W1_1, b_1):
    src = edge_index[0]
    dst = edge_index[1]
    a0, z0 = _tc_pre(y, W0_0.T, W1_0.T, b_0.reshape(1, D))
    agg0 = _sc_agg(z0, src, dst, edge_weight)
    a1, z1 = _tc_mid(a0, agg0, W0_1.T, W1_1.T, b_1.reshape(1, D))
    agg1 = _sc_agg(z1, src, dst, edge_weight)
    return _tc_post(a1, agg1)
